# Initial kernel scaffold; baseline (speedup 1.0000x reference)
import jax
import jax.numpy as jnp
from jax.experimental import pallas as pl


def kernel(x, edge_index, edge_weight, batch, W1, b1, W2, b2, Wfc, bfc):
    raise NotImplementedError("write your pallas kernel here")


# speedup vs baseline: 15.9987x; 15.9987x over previous
import functools

import jax
import jax.numpy as jnp
from jax import lax
from jax.experimental import pallas as pl
from jax.experimental.pallas import tpu as pltpu
from jax.experimental.pallas import tpu_sc as plsc

N = 10000
E = 320000
D_IN = 128
D_H = 64
D_OUT = 128
G = 64

NP = 10240
NC, NS, L = 2, 16, 16
NW = NC * NS
CH = 128
E_TOT = E + N
NCHUNK = 81
EP = NW * NCHUNK * CH
ROWS_PER_TILE = NP // NS


def _tile_id():
    return lax.axis_index("c"), lax.axis_index("s")


def _zero_fill(buf, nrow, ncol):
    z = jnp.zeros((L,), jnp.float32)
    for r in range(nrow):
        for q in range(ncol // L):
            buf[r, pl.ds(q * L, L)] = z


def _sc_deg_body(col_hbm, ew_hbm, degp_hbm,
                 colbuf, ewflat, padbuf, acc):
    c, s = _tile_id()
    wid = s * NC + c
    _zero_fill(padbuf, CH, L)
    for k in range(ROWS_PER_TILE // CH):
        pltpu.sync_copy(padbuf, acc.at[pl.ds(s * ROWS_PER_TILE + k * CH, CH)])
    plsc.subcore_barrier()

    pltpu.sync_copy(col_hbm.at[wid], colbuf)
    pltpu.sync_copy(ew_hbm.at[wid], ewflat)

    def chunk(i, carry):
        for e in range(CH):
            b = plsc.load_gather(ewflat, [jnp.full((L,), i * CH + e, jnp.int32)])
            padbuf[e, :] = b
        pltpu.sync_copy(padbuf, acc.at[colbuf.at[i]], add=True)
        return carry

    lax.fori_loop(0, NCHUNK, chunk, 0)
    plsc.subcore_barrier()

    for k in range(ROWS_PER_TILE // CH):
        base = s * ROWS_PER_TILE + k * CH
        pltpu.sync_copy(acc.at[pl.ds(base, CH)], padbuf)
        pltpu.sync_copy(padbuf, degp_hbm.at[c, pl.ds(base, CH)])


def _sc_deg(col3, ew2):
    mesh = plsc.VectorSubcoreMesh(core_axis_name="c", subcore_axis_name="s")
    return pl.kernel(
        _sc_deg_body,
        out_type=jax.ShapeDtypeStruct((NC, NP, L), jnp.float32),
        mesh=mesh,
        compiler_params=pltpu.CompilerParams(needs_layout_passes=False, use_tc_tiling_on_sc=False),
        scratch_types=[
            pltpu.VMEM((NCHUNK, CH), jnp.int32),
            pltpu.VMEM((NCHUNK * CH,), jnp.float32),
            pltpu.VMEM((CH, L), jnp.float32),
            pltpu.VMEM_SHARED((NP, L), jnp.float32),
        ],
    )(col3, ew2)


def _msg_common(i, rowbuf, colbuf, normbuf, xw_hbm, msgbuf, acc, sem):
    cp = pltpu.async_copy(xw_hbm.at[rowbuf.at[i]], msgbuf, sem)
    cp.wait()
    for e in range(CH):
        b = plsc.load_gather(normbuf, [jnp.full((L,), e, jnp.int32)])
        for q in range(D_H // L):
            msgbuf[e, pl.ds(q * L, L)] = msgbuf[e, pl.ds(q * L, L)] * b
    pltpu.sync_copy(msgbuf, acc.at[colbuf.at[i]], add=True)


def _zero_acc_and_sync(zbuf, acc, s):
    _zero_fill(zbuf, CH, D_H)
    for k in range(ROWS_PER_TILE // CH):
        pltpu.sync_copy(zbuf, acc.at[pl.ds(s * ROWS_PER_TILE + k * CH, CH)])
    plsc.subcore_barrier()


def _emit_parts(zbuf, acc, parts_hbm, c, s):
    plsc.subcore_barrier()
    for k in range(ROWS_PER_TILE // CH):
        base = s * ROWS_PER_TILE + k * CH
        pltpu.sync_copy(acc.at[pl.ds(base, CH)], zbuf)
        pltpu.sync_copy(zbuf, parts_hbm.at[c, pl.ds(base, CH)])


def _sc_msg1_body(row_hbm, col_hbm, ew_hbm, dis_hbm, xw_hbm,
                  parts_hbm, norm_hbm,
                  rowbuf, colbuf, ewbuf, disbuf, normbuf, msgbuf, zbuf, acc, sem):
    c, s = _tile_id()
    wid = s * NC + c
    _zero_acc_and_sync(zbuf, acc, s)
    pltpu.sync_copy(row_hbm.at[wid], rowbuf)
    pltpu.sync_copy(col_hbm.at[wid], colbuf)
    pltpu.sync_copy(ew_hbm.at[wid], ewbuf)
    pltpu.sync_copy(dis_hbm, disbuf)

    def chunk(i, carry):
        for g in range(CH // L):
            r16 = rowbuf[i, pl.ds(g * L, L)]
            c16 = colbuf[i, pl.ds(g * L, L)]
            w16 = ewbuf[i, pl.ds(g * L, L)]
            n16 = plsc.load_gather(disbuf, [r16]) * w16 * plsc.load_gather(disbuf, [c16])
            normbuf[pl.ds(g * L, L)] = n16
        pltpu.sync_copy(normbuf, norm_hbm.at[wid].at[i])
        _msg_common(i, rowbuf, colbuf, normbuf, xw_hbm, msgbuf, acc, sem)
        return carry

    lax.fori_loop(0, NCHUNK, chunk, 0)
    _emit_parts(zbuf, acc, parts_hbm, c, s)


def _sc_msg2_body(row_hbm, col_hbm, norm_in_hbm, xw_hbm,
                  parts_hbm,
                  rowbuf, colbuf, normfull, normbuf, msgbuf, zbuf, acc, sem):
    c, s = _tile_id()
    wid = s * NC + c
    _zero_acc_and_sync(zbuf, acc, s)
    pltpu.sync_copy(row_hbm.at[wid], rowbuf)
    pltpu.sync_copy(col_hbm.at[wid], colbuf)
    pltpu.sync_copy(norm_in_hbm.at[wid], normfull)

    def chunk(i, carry):
        for g in range(CH // L):
            normbuf[pl.ds(g * L, L)] = normfull[i, pl.ds(g * L, L)]
        _msg_common(i, rowbuf, colbuf, normbuf, xw_hbm, msgbuf, acc, sem)
        return carry

    lax.fori_loop(0, NCHUNK, chunk, 0)
    _emit_parts(zbuf, acc, parts_hbm, c, s)


def _sc_msg1(row3, col3, ew3, dis, xw):
    mesh = plsc.VectorSubcoreMesh(core_axis_name="c", subcore_axis_name="s")
    return pl.kernel(
        _sc_msg1_body,
        out_type=(jax.ShapeDtypeStruct((NC, NP, D_H), jnp.float32),
                  jax.ShapeDtypeStruct((NW, NCHUNK, CH), jnp.float32)),
        mesh=mesh,
        compiler_params=pltpu.CompilerParams(needs_layout_passes=False, use_tc_tiling_on_sc=False),
        scratch_types=[
            pltpu.VMEM((NCHUNK, CH), jnp.int32),
            pltpu.VMEM((NCHUNK, CH), jnp.int32),
            pltpu.VMEM((NCHUNK, CH), jnp.float32),
            pltpu.VMEM((NP,), jnp.float32),
            pltpu.VMEM((CH,), jnp.float32),
            pltpu.VMEM((CH, D_H), jnp.float32),
            pltpu.VMEM((CH, D_H), jnp.float32),
            pltpu.VMEM_SHARED((NP, D_H), jnp.float32),
            pltpu.SemaphoreType.DMA,
        ],
    )(row3, col3, ew3, dis, xw)


def _sc_msg2(row3, col3, norm3, xw):
    mesh = plsc.VectorSubcoreMesh(core_axis_name="c", subcore_axis_name="s")
    return pl.kernel(
        _sc_msg2_body,
        out_type=jax.ShapeDtypeStruct((NC, NP, D_H), jnp.float32),
        mesh=mesh,
        compiler_params=pltpu.CompilerParams(needs_layout_passes=False, use_tc_tiling_on_sc=False),
        scratch_types=[
            pltpu.VMEM((NCHUNK, CH), jnp.int32),
            pltpu.VMEM((NCHUNK, CH), jnp.int32),
            pltpu.VMEM((NCHUNK, CH), jnp.float32),
            pltpu.VMEM((CH,), jnp.float32),
            pltpu.VMEM((CH, D_H), jnp.float32),
            pltpu.VMEM((CH, D_H), jnp.float32),
            pltpu.VMEM_SHARED((NP, D_H), jnp.float32),
            pltpu.SemaphoreType.DMA,
        ],
    )(row3, col3, norm3, xw)


def _tc_mm_body(x_ref, w_ref, o_ref):
    o_ref[...] = jax.lax.dot_general(
        x_ref[...], w_ref[...], (((1,), (0,)), ((), ())),
        preferred_element_type=jnp.float32,
        precision=jax.lax.Precision.HIGHEST)


def _tc_mm(x, w):
    return pl.pallas_call(
        _tc_mm_body,
        out_shape=jax.ShapeDtypeStruct((x.shape[0], w.shape[1]), jnp.float32),
    )(x, w)


def _tc_dis_body(degp_ref, dis_ref):
    deg = degp_ref[0, :, 0] + degp_ref[1, :, 0]
    safe = jnp.where(deg > 0.0, deg, 1.0)
    dis_ref[...] = jnp.where(deg > 0.0, jax.lax.rsqrt(safe), 0.0)


def _tc_dis(degp):
    return pl.pallas_call(
        _tc_dis_body,
        out_shape=jax.ShapeDtypeStruct((NP,), jnp.float32),
    )(degp)


def _tc_combine1_body(p_ref, b_ref, w2_ref, xw2_ref):
    h = p_ref[0] + p_ref[1] + b_ref[...][None, :]
    h = jnp.maximum(h, 0.0)
    rows = jax.lax.broadcasted_iota(jnp.int32, (NP, D_H), 0)
    h = jnp.where(rows < N, h, 0.0)
    xw2_ref[...] = jax.lax.dot_general(
        h, w2_ref[...], (((1,), (0,)), ((), ())),
        preferred_element_type=jnp.float32,
        precision=jax.lax.Precision.HIGHEST)


def _tc_combine1(parts, b1, W2):
    return pl.pallas_call(
        _tc_combine1_body,
        out_shape=jax.ShapeDtypeStruct((NP, D_H), jnp.float32),
    )(parts, b1, W2)


def _tc_pool_body(p_ref, b2_ref, batch_ref, wfc_ref, bfc_ref, o_ref):
    h2 = p_ref[0] + p_ref[1] + b2_ref[...][None, :]
    seg = batch_ref[...]
    gids = jax.lax.broadcasted_iota(jnp.int32, (G, NP), 0)
    onehot = (seg[None, :] == gids).astype(jnp.float32)
    s = jax.lax.dot_general(
        onehot, h2, (((1,), (0,)), ((), ())),
        preferred_element_type=jnp.float32,
        precision=jax.lax.Precision.HIGHEST)
    cnt = jnp.sum(onehot, axis=1)
    pooled = s / jnp.clip(cnt, 1.0)[:, None]
    o_ref[...] = jax.lax.dot_general(
        pooled, wfc_ref[...], (((1,), (0,)), ((), ())),
        preferred_element_type=jnp.float32,
        precision=jax.lax.Precision.HIGHEST) + bfc_ref[...][None, :]


def _tc_pool(parts, b2, batch_pad, Wfc, bfc):
    return pl.pallas_call(
        _tc_pool_body,
        out_shape=jax.ShapeDtypeStruct((G, D_OUT), jnp.float32),
    )(parts, b2, batch_pad, Wfc, bfc)


def kernel(x, edge_index, edge_weight, batch, W1, b1, W2, b2, Wfc, bfc):
    pad_e = EP - E_TOT
    loops = jnp.arange(N, dtype=jnp.int32)
    padv = jnp.full((pad_e,), NP - 1, jnp.int32)
    rows = jnp.concatenate([edge_index[0], loops, padv]).reshape(NW, NCHUNK, CH)
    cols = jnp.concatenate([edge_index[1], loops, padv]).reshape(NW, NCHUNK, CH)
    ews = jnp.concatenate([
        edge_weight, jnp.ones((N,), jnp.float32), jnp.zeros((pad_e,), jnp.float32)
    ]).reshape(NW, NCHUNK, CH)
    xpad = jnp.pad(x, ((0, NP - N), (0, 0)))
    batch_pad = jnp.pad(batch, (0, NP - N), constant_values=G)

    xw1 = _tc_mm(xpad, W1)
    degp = _sc_deg(cols, ews.reshape(NW, NCHUNK * CH))
    dis = _tc_dis(degp)
    parts1, norm3 = _sc_msg1(rows, cols, ews, dis, xw1)
    xw2 = _tc_combine1(parts1, b1, W2)
    parts2 = _sc_msg2(rows, cols, norm3, xw2)
    return _tc_pool(parts2, b2, batch_pad, Wfc, bfc)
